# baseline (device time: 46662 ns/iter reference)
import os

import jax
import jax.numpy as jnp
from jax import lax
from jax.experimental import pallas as pl
from jax.experimental.pallas import tpu as pltpu

N_DEV = 16

_ABLATE = os.environ.get("ABLATE", "")
_NOAMAX = os.environ.get("NOAMAX", "") == "1"


def kernel(x, w_mat):
    m, k_per = x.shape
    k, n = w_mat.shape
    m_blk = m // N_DEV

    def body(x_ref, w_hbm_ref, out_ref, xb_ref, xg_ref, wstage_ref,
             wb_ref, amax_ref, send_sems, recv_sems, send2_sems,
             recv2_sems, wdma_sems):
        my_i = lax.axis_index("i")

        xb_ref[:, :] = x_ref[:, :].astype(jnp.bfloat16)
        wdma = []
        for d in range(N_DEV):
            s = (my_i - d) % N_DEV
            cp = pltpu.make_async_copy(
                w_hbm_ref.at[pl.ds(s * k_per, k_per), :],
                wstage_ref.at[d % 4],
                wdma_sems.at[d],
            )
            if d < 4:
                cp.start()
            wdma.append(cp)

        if _ABLATE != "compute":
            barrier_sem = pltpu.get_barrier_semaphore()
            for d in range(1, N_DEV):
                pl.semaphore_signal(
                    barrier_sem, inc=1,
                    device_id=((my_i + d) % N_DEV,),
                    device_id_type=pl.DeviceIdType.MESH,
                )
            pl.semaphore_wait(barrier_sem, N_DEV - 1)

        a2a = []
        if _ABLATE != "compute":
            for d in range(1, N_DEV):
                t = (my_i + d) % N_DEV
                rdma = pltpu.make_async_remote_copy(
                    src_ref=xb_ref.at[pl.ds(t * m_blk, m_blk), :],
                    dst_ref=xg_ref.at[:, pl.ds(d * k_per, k_per)],
                    send_sem=send_sems.at[d],
                    recv_sem=recv_sems.at[d],
                    device_id=(t,),
                    device_id_type=pl.DeviceIdType.MESH,
                )
                rdma.start()
                a2a.append(rdma)

        xg_ref[:, 0:k_per] = xb_ref[pl.ds(my_i * m_blk, m_blk), :]
        for d in range(N_DEV):
            wdma[d].wait()
            wb_ref[pl.ds(d * k_per, k_per), :] = (
                wstage_ref[d % 4].astype(jnp.bfloat16)
            )
            if d + 4 < N_DEV:
                wdma[d + 4].start()

        if _ABLATE == "comm":
            out_ref[:, :] = jnp.zeros((m_blk, n), jnp.float32)
        for g in range(4):
            if _ABLATE != "compute":
                for d in range(4 * g, 4 * g + 4):
                    if d > 0:
                        a2a[d - 1].wait()
            if _ABLATE != "comm":
                blk = pl.ds(4 * g * k_per, 4 * k_per)
                acc = jnp.dot(
                    xg_ref[:, blk],
                    wb_ref[blk, :],
                    preferred_element_type=jnp.float32,
                )
                if g == 0:
                    out_ref[:, :] = acc
                else:
                    out_ref[:, :] = out_ref[:, :] + acc

        local_amax = jnp.max(jnp.abs(out_ref[:, :]))
        amax_ref[0, :] = jnp.full((128,), local_amax, jnp.float32)
        if _ABLATE != "compute" and not _NOAMAX:
            ax = []
            for d in range(1, N_DEV):
                t = (my_i + d) % N_DEV
                rdma = pltpu.make_async_remote_copy(
                    src_ref=amax_ref.at[0],
                    dst_ref=amax_ref.at[d],
                    send_sem=send2_sems.at[d],
                    recv_sem=recv2_sems.at[d],
                    device_id=(t,),
                    device_id_type=pl.DeviceIdType.MESH,
                )
                rdma.start()
                ax.append(rdma)
            for rdma in ax:
                rdma.wait()

        gmax = jnp.max(amax_ref[:, :])
        scale = gmax / 127.0
        q = jnp.clip(jnp.round(out_ref[:, :] / scale), -127.0, 127.0)
        out_ref[:, :] = q * scale

    return pl.pallas_call(
        body,
        out_shape=jax.ShapeDtypeStruct((m_blk, n), jnp.float32),
        in_specs=[
            pl.BlockSpec(memory_space=pltpu.VMEM),
            pl.BlockSpec(memory_space=pltpu.MemorySpace.HBM),
        ],
        out_specs=pl.BlockSpec(memory_space=pltpu.VMEM),
        scratch_shapes=[
            pltpu.VMEM((m, k_per), jnp.bfloat16),
            pltpu.VMEM((m_blk, N_DEV * k_per), jnp.bfloat16),
            pltpu.VMEM((4, k_per, n), jnp.float32),
            pltpu.VMEM((k, n), jnp.bfloat16),
            pltpu.VMEM((N_DEV, 128), jnp.float32),
            pltpu.SemaphoreType.DMA((N_DEV,)),
            pltpu.SemaphoreType.DMA((N_DEV,)),
            pltpu.SemaphoreType.DMA((N_DEV,)),
            pltpu.SemaphoreType.DMA((N_DEV,)),
            pltpu.SemaphoreType.DMA((N_DEV,)),
        ],
        compiler_params=pltpu.CompilerParams(
            collective_id=None if _ABLATE == "compute" else 0,
            vmem_limit_bytes=100 * 1024 * 1024,
        ),
    )(x, w_mat)


# device time: 46368 ns/iter; 1.0063x vs baseline; 1.0063x over previous
import os

import jax
import jax.numpy as jnp
from jax import lax
from jax.experimental import pallas as pl
from jax.experimental.pallas import tpu as pltpu

N_DEV = 16

_ABLATE = os.environ.get("ABLATE", "")
_NOAMAX = os.environ.get("NOAMAX", "") == "1"


def kernel(x, w_mat):
    m, k_per = x.shape
    k, n = w_mat.shape
    m_blk = m // N_DEV

    def body(x_ref, w_hbm_ref, out_ref, xb_ref, xg_ref, wstage_ref,
             wb_ref, amax_ref, send_sems, recv_sems, send2_sems,
             recv2_sems, wdma_sems):
        my_i = lax.axis_index("i")

        if _ABLATE != "compute":
            barrier_sem = pltpu.get_barrier_semaphore()
            for d in range(1, N_DEV):
                pl.semaphore_signal(
                    barrier_sem, inc=1,
                    device_id=((my_i + d) % N_DEV,),
                    device_id_type=pl.DeviceIdType.MESH,
                )

        xb_ref[:, :] = x_ref[:, :].astype(jnp.bfloat16)
        wdma = []
        for d in range(N_DEV):
            s = (my_i - d) % N_DEV
            cp = pltpu.make_async_copy(
                w_hbm_ref.at[pl.ds(s * k_per, k_per), :],
                wstage_ref.at[d % 4],
                wdma_sems.at[d],
            )
            if d < 4:
                cp.start()
            wdma.append(cp)

        if _ABLATE != "compute":
            pl.semaphore_wait(barrier_sem, N_DEV - 1)

        a2a = []
        if _ABLATE != "compute":
            for d in range(1, N_DEV):
                t = (my_i + d) % N_DEV
                rdma = pltpu.make_async_remote_copy(
                    src_ref=xb_ref.at[pl.ds(t * m_blk, m_blk), :],
                    dst_ref=xg_ref.at[:, pl.ds(d * k_per, k_per)],
                    send_sem=send_sems.at[d],
                    recv_sem=recv_sems.at[d],
                    device_id=(t,),
                    device_id_type=pl.DeviceIdType.MESH,
                )
                rdma.start()
                a2a.append(rdma)

        xg_ref[:, 0:k_per] = xb_ref[pl.ds(my_i * m_blk, m_blk), :]
        for d in range(N_DEV):
            wdma[d].wait()
            wb_ref[pl.ds(d * k_per, k_per), :] = (
                wstage_ref[d % 4].astype(jnp.bfloat16)
            )
            if d + 4 < N_DEV:
                wdma[d + 4].start()

        if _ABLATE == "comm":
            out_ref[:, :] = jnp.zeros((m_blk, n), jnp.float32)
        for g in range(4):
            if _ABLATE != "compute":
                for d in range(4 * g, 4 * g + 4):
                    if d > 0:
                        a2a[d - 1].wait()
            if _ABLATE != "comm":
                blk = pl.ds(4 * g * k_per, 4 * k_per)
                acc = jnp.dot(
                    xg_ref[:, blk],
                    wb_ref[blk, :],
                    preferred_element_type=jnp.float32,
                )
                if g == 0:
                    out_ref[:, :] = acc
                else:
                    out_ref[:, :] = out_ref[:, :] + acc

        local_amax = jnp.max(jnp.abs(out_ref[:, :]))
        amax_ref[0, :] = jnp.full((128,), local_amax, jnp.float32)
        if _ABLATE != "compute" and not _NOAMAX:
            ax = []
            for d in range(1, N_DEV):
                t = (my_i + d) % N_DEV
                rdma = pltpu.make_async_remote_copy(
                    src_ref=amax_ref.at[0],
                    dst_ref=amax_ref.at[d],
                    send_sem=send2_sems.at[d],
                    recv_sem=recv2_sems.at[d],
                    device_id=(t,),
                    device_id_type=pl.DeviceIdType.MESH,
                )
                rdma.start()
                ax.append(rdma)
            for rdma in ax:
                rdma.wait()

        gmax = jnp.max(amax_ref[:, :])
        scale = gmax / 127.0
        q = jnp.clip(jnp.round(out_ref[:, :] / scale), -127.0, 127.0)
        out_ref[:, :] = q * scale

    return pl.pallas_call(
        body,
        out_shape=jax.ShapeDtypeStruct((m_blk, n), jnp.float32),
        in_specs=[
            pl.BlockSpec(memory_space=pltpu.VMEM),
            pl.BlockSpec(memory_space=pltpu.MemorySpace.HBM),
        ],
        out_specs=pl.BlockSpec(memory_space=pltpu.VMEM),
        scratch_shapes=[
            pltpu.VMEM((m, k_per), jnp.bfloat16),
            pltpu.VMEM((m_blk, N_DEV * k_per), jnp.bfloat16),
            pltpu.VMEM((4, k_per, n), jnp.float32),
            pltpu.VMEM((k, n), jnp.bfloat16),
            pltpu.VMEM((N_DEV, 128), jnp.float32),
            pltpu.SemaphoreType.DMA((N_DEV,)),
            pltpu.SemaphoreType.DMA((N_DEV,)),
            pltpu.SemaphoreType.DMA((N_DEV,)),
            pltpu.SemaphoreType.DMA((N_DEV,)),
            pltpu.SemaphoreType.DMA((N_DEV,)),
        ],
        compiler_params=pltpu.CompilerParams(
            collective_id=None if _ABLATE == "compute" else 0,
            vmem_limit_bytes=100 * 1024 * 1024,
        ),
    )(x, w_mat)


# device time: 45094 ns/iter; 1.0348x vs baseline; 1.0283x over previous
import os

import jax
import jax.numpy as jnp
from jax import lax
from jax.experimental import pallas as pl
from jax.experimental.pallas import tpu as pltpu

N_DEV = 16

_ABLATE = os.environ.get("ABLATE", "")
_NOAMAX = os.environ.get("NOAMAX", "") == "1"


def kernel(x, w_mat):
    m, k_per = x.shape
    k, n = w_mat.shape
    m_blk = m // N_DEV

    def body(x_ref, w_hbm_ref, out_ref, xb_ref, xg_ref, wstage_ref,
             wb_ref, send_sems, recv_sems, wdma_sems, amax_sems):
        my_i = lax.axis_index("i")

        if _ABLATE != "compute":
            barrier_sem = pltpu.get_barrier_semaphore()
            for d in range(1, N_DEV):
                pl.semaphore_signal(
                    barrier_sem, inc=1,
                    device_id=((my_i + d) % N_DEV,),
                    device_id_type=pl.DeviceIdType.MESH,
                )

        xb_ref[:, :] = x_ref[:, :].astype(jnp.bfloat16)
        wdma = []
        for d in range(N_DEV):
            s = (my_i - d) % N_DEV
            cp = pltpu.make_async_copy(
                w_hbm_ref.at[pl.ds(s * k_per, k_per), :],
                wstage_ref.at[d % 4],
                wdma_sems.at[d],
            )
            if d < 4:
                cp.start()
            wdma.append(cp)

        if _ABLATE != "compute":
            pl.semaphore_wait(barrier_sem, N_DEV - 1)

        a2a = []
        if _ABLATE != "compute":
            for d in range(1, N_DEV):
                t = (my_i + d) % N_DEV
                rdma = pltpu.make_async_remote_copy(
                    src_ref=xb_ref.at[pl.ds(t * m_blk, m_blk), :],
                    dst_ref=xg_ref.at[:, pl.ds(d * k_per, k_per)],
                    send_sem=send_sems.at[d],
                    recv_sem=recv_sems.at[d],
                    device_id=(t,),
                    device_id_type=pl.DeviceIdType.MESH,
                )
                rdma.start()
                a2a.append(rdma)

        xg_ref[:, 0:k_per] = xb_ref[pl.ds(my_i * m_blk, m_blk), :]
        for d in range(N_DEV):
            wdma[d].wait()
            wb_ref[pl.ds(d * k_per, k_per), :] = (
                wstage_ref[d % 4].astype(jnp.bfloat16)
            )
            if d + 4 < N_DEV:
                wdma[d + 4].start()

        if _ABLATE == "comm":
            out_ref[:, :] = jnp.zeros((m_blk, n), jnp.float32)
        for g in range(4):
            if _ABLATE != "compute":
                for d in range(4 * g, 4 * g + 4):
                    if d > 0:
                        a2a[d - 1].wait()
            if _ABLATE != "comm":
                blk = pl.ds(4 * g * k_per, 4 * k_per)
                acc = jnp.dot(
                    xg_ref[:, blk],
                    wb_ref[blk, :],
                    preferred_element_type=jnp.float32,
                )
                if g == 0:
                    out_ref[:, :] = acc
                else:
                    out_ref[:, :] = out_ref[:, :] + acc

        local_amax = jnp.max(jnp.abs(out_ref[:, :]))
        enc = lax.shift_right_logical(
            lax.bitcast_convert_type(local_amax, jnp.int32), 8
        )
        gmax = lax.bitcast_convert_type(
            lax.shift_left(enc, 8), jnp.float32
        )
        if _ABLATE != "compute" and not _NOAMAX:
            for o in range(1, N_DEV):
                t = (my_i + o) % N_DEV
                pl.semaphore_signal(
                    amax_sems.at[o], inc=enc,
                    device_id=(t,),
                    device_id_type=pl.DeviceIdType.MESH,
                )
            for o in range(1, N_DEV):
                pl.semaphore_wait(amax_sems.at[o], 1, decrement=False)
                v = pl.semaphore_read(amax_sems.at[o])
                pl.semaphore_wait(amax_sems.at[o], v)
                gmax = jnp.maximum(
                    gmax,
                    lax.bitcast_convert_type(
                        lax.shift_left(v, 8), jnp.float32
                    ),
                )

        scale = gmax / 127.0
        q = jnp.clip(jnp.round(out_ref[:, :] / scale), -127.0, 127.0)
        out_ref[:, :] = q * scale

    return pl.pallas_call(
        body,
        out_shape=jax.ShapeDtypeStruct((m_blk, n), jnp.float32),
        in_specs=[
            pl.BlockSpec(memory_space=pltpu.VMEM),
            pl.BlockSpec(memory_space=pltpu.MemorySpace.HBM),
        ],
        out_specs=pl.BlockSpec(memory_space=pltpu.VMEM),
        scratch_shapes=[
            pltpu.VMEM((m, k_per), jnp.bfloat16),
            pltpu.VMEM((m_blk, N_DEV * k_per), jnp.bfloat16),
            pltpu.VMEM((4, k_per, n), jnp.float32),
            pltpu.VMEM((k, n), jnp.bfloat16),
            pltpu.SemaphoreType.DMA((N_DEV,)),
            pltpu.SemaphoreType.DMA((N_DEV,)),
            pltpu.SemaphoreType.DMA((N_DEV,)),
            pltpu.SemaphoreType.REGULAR((N_DEV,)),
        ],
        compiler_params=pltpu.CompilerParams(
            collective_id=None if _ABLATE == "compute" else 0,
            vmem_limit_bytes=100 * 1024 * 1024,
        ),
    )(x, w_mat)
